# baseline (speedup 1.0000x reference)
import functools

import jax
import jax.numpy as jnp
from jax import lax
from jax.experimental import pallas as pl
from jax.experimental.pallas import tpu as pltpu
from jax.experimental.pallas import tpu_sc as plsc


def _tc_scores_and_idx(v_i2, v_j2, neg2, second, ctx_pad):
    V, D = second.shape
    VP = ctx_pad.shape[0]
    K = neg2.shape[0]
    B = v_i2.shape[1]
    ND = 1 + K

    def body(vi_ref, vj_ref, neg_ref, sec_ref, ctx_ref,
             m_ref, mini_ref, off_ref):
        m_ref[...] = lax.dot_general(
            sec_ref[...], ctx_ref[...], (((1,), (1,)), ((), ())),
            preferred_element_type=jnp.float32)
        vi = vi_ref[...]
        base = vi * (VP // 16)
        vj = vj_ref[...]
        neg = neg_ref[...]
        mini_ref[0:1, :] = base + vj // 16
        mini_ref[1:ND, :] = jnp.broadcast_to(base, (K, B)) + neg // 16
        off_ref[0:1, :] = vj % 16
        off_ref[1:ND, :] = neg % 16

    return pl.pallas_call(
        body,
        out_shape=(
            jax.ShapeDtypeStruct((V, VP), jnp.float32),
            jax.ShapeDtypeStruct((ND, B), jnp.int32),
            jax.ShapeDtypeStruct((ND, B), jnp.int32),
        ),
    )(v_i2, v_j2, neg2, second, ctx_pad)


def _sc_gather(m16, idx_packed, NW, BW, ND):
    NC = NW // 16

    mesh = plsc.VectorSubcoreMesh(core_axis_name="c", subcore_axis_name="s",
                                  num_cores=NC)

    @functools.partial(
        pl.kernel,
        mesh=mesh,
        out_type=jax.ShapeDtypeStruct((NW * ND, BW, 16), jnp.float32),
        compiler_params=pltpu.CompilerParams(use_tc_tiling_on_sc=False),
        scratch_types=[
            pltpu.VMEM((ND, BW), jnp.int32),
            pltpu.VMEM((ND, BW, 16), jnp.float32),
            pltpu.SemaphoreType.DMA,
        ],
    )
    def k(m_hbm, idx_hbm, out_hbm, idx_v, rows_v, sem):
        wid = lax.axis_index("s") * NC + lax.axis_index("c")

        pltpu.sync_copy(idx_hbm.at[pl.ds(wid * ND, ND)], idx_v)

        cps = [
            pltpu.async_copy(m_hbm.at[idx_v.at[d]], rows_v.at[d], sem)
            for d in range(ND)
        ]
        for cp in cps:
            cp.wait()

        pltpu.sync_copy(rows_v, out_hbm.at[pl.ds(wid * ND, ND)])

    return k(m16, idx_packed)


def _tc_finalize(x, off8, batch, num_dots, block):
    R, C = x.shape
    L = 16
    G = C // L

    def body(x_ref, off_ref, o_ref):
        xs = x_ref[...]
        off = off_ref[...]
        offc = jnp.reshape(
            jnp.broadcast_to(off[:, :, None], (R, G, L)), (R, C))
        lane = lax.broadcasted_iota(jnp.int32, (R, C), 1) % L
        picked = jnp.where(lane == offc, xs, 0.0)
        col = lax.broadcasted_iota(jnp.int32, (C, G), 0)
        grp = lax.broadcasted_iota(jnp.int32, (C, G), 1)
        a = (col // L == grp).astype(jnp.float32)
        s = jnp.dot(picked, a, preferred_element_type=jnp.float32)

        row = lax.broadcasted_iota(jnp.int32, (R, G), 0)
        g = lax.broadcasted_iota(jnp.int32, (R, G), 1)
        d = ((row * G + g) // block) % num_dots
        v = jnp.where(d == 0, s, -s)
        acc = jnp.minimum(v, 0.0) - jnp.log1p(jnp.exp(-jnp.abs(v)))
        o_ref[...] = jnp.broadcast_to(-(jnp.sum(acc) / batch), (1, 1))

    return pl.pallas_call(
        body,
        out_shape=jax.ShapeDtypeStruct((1, 1), jnp.float32),
    )(x, off8)


def kernel(nodeindex, v_i, v_j, negsamples, first_embeddings,
           second_embeddings, context_embeddings):
    del nodeindex, first_embeddings
    B = v_i.shape[0]
    K = negsamples.shape[0]
    V, D = second_embeddings.shape
    ND = 1 + K
    NW = 16
    BW = B // NW
    VP = 1024

    ctx_pad = jnp.pad(context_embeddings, ((0, VP - V), (0, 0)))
    m, mini, off = _tc_scores_and_idx(
        v_i.reshape(1, B), v_j.reshape(1, B), negsamples,
        second_embeddings, ctx_pad)

    def pack(t):
        return (t.reshape(ND, NW, BW).transpose(1, 0, 2)
                .reshape(NW * ND, BW))

    rows = _sc_gather(m.reshape((V * VP) // 16, 16), pack(mini),
                      NW, BW, ND)
    x = rows.reshape((NW * ND * BW * 16) // 128, 128)
    off8 = pack(off).reshape((NW * ND * BW) // 8, 8)
    loss = _tc_finalize(x, off8, B, ND, BW)
    return loss[0, 0]


# speedup vs baseline: 1.0367x; 1.0367x over previous
import functools

import jax
import jax.numpy as jnp
from jax import lax
from jax.experimental import pallas as pl
from jax.experimental.pallas import tpu as pltpu
from jax.experimental.pallas import tpu_sc as plsc


def _sc_dots(idx_packed, second, context, NW, BW, K, L):
    D = second.shape[1]
    NC = NW // 16
    NCH = D // L
    NI = 2 + K
    ND = 1 + K

    mesh = plsc.VectorSubcoreMesh(core_axis_name="c", subcore_axis_name="s",
                                  num_cores=NC)

    @functools.partial(
        pl.kernel,
        mesh=mesh,
        out_type=jax.ShapeDtypeStruct((NW * ND, BW, L), jnp.float32),
        compiler_params=pltpu.CompilerParams(use_tc_tiling_on_sc=False),
        scratch_types=[
            pltpu.VMEM((NI, BW), jnp.int32),
            pltpu.VMEM((BW, D), jnp.float32),
            pltpu.VMEM((BW, D), jnp.float32),
            pltpu.VMEM((2, BW, D), jnp.float32),
            pltpu.VMEM((ND, BW, L), jnp.float32),
            pltpu.SemaphoreType.DMA,
        ],
    )
    def k(idx_hbm, second_hbm, context_hbm, out_hbm,
          idx_v, vi_rows, vj_rows, neg_rows, out_v, sem):
        wid = lax.axis_index("s") * NC + lax.axis_index("c")

        pltpu.sync_copy(idx_hbm.at[pl.ds(wid * NI, NI)], idx_v)

        cps = [
            pltpu.async_copy(second_hbm.at[idx_v.at[0]], vi_rows, sem),
            pltpu.async_copy(context_hbm.at[idx_v.at[1]], vj_rows, sem),
        ]
        for kk in range(min(2, K)):
            cps.append(
                pltpu.async_copy(context_hbm.at[idx_v.at[2 + kk]],
                                 neg_rows.at[kk], sem))
        cps[0].wait()
        cps[1].wait()

        @plsc.parallel_loop(0, BW, unroll=8)
        def pos_body(g):
            acc = vi_rows[g, pl.ds(0, L)] * vj_rows[g, pl.ds(0, L)]
            for c in range(1, NCH):
                acc = acc + (vi_rows[g, pl.ds(c * L, L)]
                             * vj_rows[g, pl.ds(c * L, L)])
            out_v[0, g, :] = acc

        for kk in range(K):
            cps[2 + kk].wait()

            @plsc.parallel_loop(0, BW, unroll=8)
            def neg_body(g, _kk=kk):
                buf = _kk % 2
                acc = (vi_rows[g, pl.ds(0, L)]
                       * neg_rows[buf, g, pl.ds(0, L)])
                for c in range(1, NCH):
                    acc = acc + (vi_rows[g, pl.ds(c * L, L)]
                                 * neg_rows[buf, g, pl.ds(c * L, L)])
                out_v[1 + _kk, g, :] = acc

            if kk + 2 < K:
                cps.append(
                    pltpu.async_copy(context_hbm.at[idx_v.at[2 + kk + 2]],
                                     neg_rows.at[kk % 2], sem))

        pltpu.sync_copy(out_v, out_hbm.at[pl.ds(wid * ND, ND)])

    return k(idx_packed, second, context)


def _tc_finalize(x, batch, num_dots, block_rows):
    R, C = x[0].shape
    L = 16
    G = C // L

    def body(*refs):
        o_ref = refs[-1]
        total = None
        for x_ref in refs[:-1]:
            xs = x_ref[...]
            col = lax.broadcasted_iota(jnp.int32, (C, G), 0)
            grp = lax.broadcasted_iota(jnp.int32, (C, G), 1)
            a = (col // L == grp).astype(jnp.float32)
            s = jnp.dot(xs, a, preferred_element_type=jnp.float32)

            row = lax.broadcasted_iota(jnp.int32, (R, G), 0)
            d = (row // block_rows) % num_dots
            v = jnp.where(d == 0, s, -s)
            acc = jnp.minimum(v, 0.0) - jnp.log1p(jnp.exp(-jnp.abs(v)))
            part = jnp.sum(acc)
            total = part if total is None else total + part
        o_ref[...] = jnp.broadcast_to(-(total / batch), (1, 1))

    return pl.pallas_call(
        body,
        out_shape=jax.ShapeDtypeStruct((1, 1), jnp.float32),
    )(*x)


def kernel(nodeindex, v_i, v_j, negsamples, first_embeddings,
           second_embeddings, context_embeddings):
    del nodeindex, first_embeddings
    B = v_i.shape[0]
    K = negsamples.shape[0]
    L = 16
    NW = 32
    BW = B // NW

    idx = jnp.concatenate(
        [v_i.reshape(1, B), v_j.reshape(1, B), negsamples], axis=0)
    idx_packed = (idx.reshape(2 + K, NW, BW)
                  .transpose(1, 0, 2)
                  .reshape(NW * (2 + K), BW))

    NH = NW // 2
    half_rows = NH * (2 + K)
    dots0 = _sc_dots(idx_packed[:half_rows], second_embeddings,
                     context_embeddings, NH, BW, K, L)
    dots1 = _sc_dots(idx_packed[half_rows:], second_embeddings,
                     context_embeddings, NH, BW, K, L)
    R = (NH * (1 + K) * BW * L) // 128
    x = (dots0.reshape(R, 128), dots1.reshape(R, 128))
    block_rows = (BW * L) // 128
    loss = _tc_finalize(x, B, 1 + K, block_rows)
    return loss[0, 0]


# speedup vs baseline: 1.3118x; 1.2654x over previous
import functools

import jax
import jax.numpy as jnp
from jax import lax
from jax.experimental import pallas as pl
from jax.experimental.pallas import tpu as pltpu
from jax.experimental.pallas import tpu_sc as plsc


def _sc_dots(v_i, v_j, neg_flat, second, context, NW, BW, K, L):
    B = v_i.shape[0]
    D = second.shape[1]
    NC = NW // 16
    NCH = D // L
    NI = 2 + K
    ND = 1 + K

    mesh = plsc.VectorSubcoreMesh(core_axis_name="c", subcore_axis_name="s",
                                  num_cores=NC)

    @functools.partial(
        pl.kernel,
        mesh=mesh,
        out_type=jax.ShapeDtypeStruct((NW * ND, BW, L), jnp.float32),
        compiler_params=pltpu.CompilerParams(use_tc_tiling_on_sc=False),
        scratch_types=[
            pltpu.VMEM((NI, BW), jnp.int32),
            pltpu.VMEM((BW, D), jnp.float32),
            pltpu.VMEM((BW, D), jnp.float32),
            pltpu.VMEM((K, BW, D), jnp.float32),
            pltpu.VMEM((ND, BW, L), jnp.float32),
            pltpu.SemaphoreType.DMA,
            pltpu.SemaphoreType.DMA,
        ],
    )
    def k(vi_hbm, vj_hbm, neg_hbm, second_hbm, context_hbm, out_hbm,
          idx_v, vi_rows, vj_rows, neg_rows, out_v, isem, sem):
        wid = lax.axis_index("s") * NC + lax.axis_index("c")
        base = wid * BW

        scps = [
            pltpu.async_copy(vi_hbm.at[pl.ds(base, BW)], idx_v.at[0], isem),
            pltpu.async_copy(vj_hbm.at[pl.ds(base, BW)], idx_v.at[1], isem),
        ]
        for kk in range(K):
            scps.append(
                pltpu.async_copy(neg_hbm.at[pl.ds(kk * B + base, BW)],
                                 idx_v.at[2 + kk], isem))
        for cp in scps:
            cp.wait()

        cps = [
            pltpu.async_copy(second_hbm.at[idx_v.at[0]], vi_rows, sem),
            pltpu.async_copy(context_hbm.at[idx_v.at[1]], vj_rows, sem),
        ]
        for kk in range(K):
            cps.append(
                pltpu.async_copy(context_hbm.at[idx_v.at[2 + kk]],
                                 neg_rows.at[kk], sem))
        cps[0].wait()
        cps[1].wait()

        @plsc.parallel_loop(0, BW, unroll=8)
        def pos_body(g):
            acc = vi_rows[g, pl.ds(0, L)] * vj_rows[g, pl.ds(0, L)]
            for c in range(1, NCH):
                acc = acc + (vi_rows[g, pl.ds(c * L, L)]
                             * vj_rows[g, pl.ds(c * L, L)])
            out_v[0, g, :] = acc

        for kk in range(K):
            cps[2 + kk].wait()

            @plsc.parallel_loop(0, BW, unroll=8)
            def neg_body(g, _kk=kk):
                acc = (vi_rows[g, pl.ds(0, L)]
                       * neg_rows[_kk, g, pl.ds(0, L)])
                for c in range(1, NCH):
                    acc = acc + (vi_rows[g, pl.ds(c * L, L)]
                                 * neg_rows[_kk, g, pl.ds(c * L, L)])
                out_v[1 + _kk, g, :] = acc

        pltpu.sync_copy(out_v, out_hbm.at[pl.ds(wid * ND, ND)])

    return k(v_i, v_j, neg_flat, second, context)


def _tc_finalize(x, batch, num_dots, block_rows):
    R, C = x.shape
    L = 16
    G = C // L

    def body(x_ref, o_ref):
        xs = x_ref[...]
        col = lax.broadcasted_iota(jnp.int32, (C, G), 0)
        grp = lax.broadcasted_iota(jnp.int32, (C, G), 1)
        a = (col // L == grp).astype(jnp.float32)
        s = jnp.dot(xs, a, preferred_element_type=jnp.float32)

        row = lax.broadcasted_iota(jnp.int32, (R, G), 0)
        d = (row // block_rows) % num_dots
        v = jnp.where(d == 0, s, -s)
        acc = jnp.minimum(v, 0.0) - jnp.log1p(jnp.exp(-jnp.abs(v)))
        o_ref[...] = jnp.broadcast_to(-(jnp.sum(acc) / batch), (1, 1))

    return pl.pallas_call(
        body,
        out_shape=jax.ShapeDtypeStruct((1, 1), jnp.float32),
    )(x)


def kernel(nodeindex, v_i, v_j, negsamples, first_embeddings,
           second_embeddings, context_embeddings):
    del nodeindex, first_embeddings
    B = v_i.shape[0]
    K = negsamples.shape[0]
    L = 16
    NW = 32
    BW = B // NW

    dots = _sc_dots(v_i, v_j, negsamples.reshape(K * B),
                    second_embeddings, context_embeddings,
                    NW, BW, K, L)
    x = dots.reshape((NW * (1 + K) * BW * L) // 128, 128)
    block_rows = (BW * L) // 128
    loss = _tc_finalize(x, B, 1 + K, block_rows)
    return loss[0, 0]
